# Initial kernel scaffold; baseline (speedup 1.0000x reference)
import jax
import jax.numpy as jnp
from jax.experimental import pallas as pl


def kernel(atom_features, bond_features, pair_indices, kernel, bias):
    raise NotImplementedError("write your pallas kernel here")


# speedup vs baseline: 2.2683x; 2.2683x over previous
import functools

import jax
import jax.numpy as jnp
from jax import lax
from jax.experimental import pallas as pl
from jax.experimental.pallas import tpu as pltpu
from jax.experimental.pallas import tpu_sc as plsc

N_NODES = 50000
ATOM_DIM = 32
BOND_DIM = 16
K_AUG = BOND_DIM + 1
M_DIM = K_AUG * ATOM_DIM

NC = 2
NT = 16
NW = NC * NT
CHUNK = 128

CPW = 25
EDGES_PW = CPW * CHUNK
E_PAD = NW * EDGES_PW

BLK = 1024
N_BLOCKS = E_PAD // BLK

HALF = 25088
OUT_PAD = 2 * HALF
DRAIN_PT = HALF // NT
ACC_ROWS = 26624
ZCHUNK = ACC_ROWS // NT
DUMMY = HALF
EPT = E_PAD // NT
SCHUNKS = 10
OUTER = EPT // (SCHUNKS * CHUNK)
SENTINEL = 1 << 30

_mesh = plsc.VectorSubcoreMesh(core_axis_name="c", subcore_axis_name="s")


@functools.partial(
    pl.kernel,
    out_type=jax.ShapeDtypeStruct((NW, CPW, CHUNK, ATOM_DIM), jnp.float32),
    mesh=_mesh,
    scratch_types=[
        pltpu.VMEM((CPW, CHUNK), jnp.int32),
        pltpu.VMEM((CPW, CHUNK, ATOM_DIM), jnp.float32),
        pltpu.SemaphoreType.DMA,
    ],
    compiler_params=pltpu.CompilerParams(use_tc_tiling_on_sc=False),
)
def _gather_rows(dst_hbm, atom_hbm, out_hbm, idx_v, rows_v, sem):
    wid = lax.axis_index("s") * NC + lax.axis_index("c")
    pltpu.sync_copy(dst_hbm.at[wid], idx_v)
    descs = [
        pltpu.async_copy(atom_hbm.at[idx_v.at[j]], rows_v.at[j], sem)
        for j in range(CPW)
    ]
    for d in descs:
        d.wait()
    pltpu.sync_copy(rows_v, out_hbm.at[wid])


def _dense_body(nb_ref, bond_ref, rmat_ref, tmat_ref, kperm_ref, out_ref):
    a = jnp.dot(bond_ref[...], rmat_ref[...], preferred_element_type=jnp.float32)
    b = jnp.dot(nb_ref[...], tmat_ref[...], preferred_element_type=jnp.float32)
    out_ref[...] = jnp.dot(a * b, kperm_ref[...], preferred_element_type=jnp.float32)


@functools.partial(
    pl.kernel,
    out_type=jax.ShapeDtypeStruct((OUT_PAD, ATOM_DIM), jnp.float32),
    mesh=_mesh,
    scratch_types=[
        pltpu.VMEM((SCHUNKS, CHUNK), jnp.int32),
        pltpu.VMEM((SCHUNKS, CHUNK), jnp.int32),
        pltpu.VMEM((SCHUNKS, CHUNK, ATOM_DIM), jnp.float32),
        pltpu.VMEM_SHARED((ACC_ROWS, ATOM_DIM), jnp.float32),
    ],
    compiler_params=pltpu.CompilerParams(use_tc_tiling_on_sc=False),
)
def _scatter_add(src_hbm, t_hbm, zero_hbm, out_hbm, sidx_v, lidx_v, tv, acc):
    c = lax.axis_index("c")
    s = lax.axis_index("s")
    pltpu.sync_copy(zero_hbm, acc.at[pl.ds(s * ZCHUNK, ZCHUNK)])
    plsc.subcore_barrier()
    base = c * HALF

    def outer(o, carry):
        pltpu.sync_copy(src_hbm.at[s, o], sidx_v)
        pltpu.sync_copy(t_hbm.at[s, o], tv)
        for r in range(SCHUNKS):
            for q in range(CHUNK // 16):
                v = sidx_v[r, pl.ds(q * 16, 16)]
                il = v - base
                ok = (il >= 0) & (il < HALF)
                lidx_v[r, pl.ds(q * 16, 16)] = jnp.where(ok, il, DUMMY)
        for r in range(SCHUNKS):
            pltpu.sync_copy(tv.at[r], acc.at[lidx_v.at[r]], add=True)
        return carry

    lax.fori_loop(0, OUTER, outer, 0)
    plsc.subcore_barrier()
    pltpu.sync_copy(
        acc.at[pl.ds(s * DRAIN_PT, DRAIN_PT)],
        out_hbm.at[pl.ds(c * HALF + s * DRAIN_PT, DRAIN_PT)],
    )


def kernel(atom_features, bond_features, pair_indices, kernel, bias):
    pi = pair_indices.astype(jnp.int32)
    src, dst = pi[:, 0], pi[:, 1]
    n_edges = src.shape[0]
    pad = E_PAD - n_edges

    dst_pad = jnp.concatenate([dst, jnp.zeros((pad,), jnp.int32)])
    src_pad = jnp.concatenate([src, jnp.full((pad,), SENTINEL, jnp.int32)])
    bond_aug = (
        jnp.zeros((E_PAD, K_AUG), jnp.float32)
        .at[:n_edges, :BOND_DIM].set(bond_features)
        .at[:n_edges, BOND_DIM].set(1.0)
    )
    kaug = jnp.concatenate([kernel, bias[None, :]], axis=0)
    kperm = kaug.reshape(K_AUG, ATOM_DIM, ATOM_DIM).transpose(0, 2, 1).reshape(
        M_DIM, ATOM_DIM
    )
    rmat = jnp.repeat(jnp.eye(K_AUG, dtype=jnp.float32), ATOM_DIM, axis=1)
    tmat = jnp.tile(jnp.eye(ATOM_DIM, dtype=jnp.float32), (1, K_AUG))

    nb = _gather_rows(
        dst_pad.reshape(NW, CPW, CHUNK), atom_features
    ).reshape(E_PAD, ATOM_DIM)

    transformed = pl.pallas_call(
        _dense_body,
        grid=(N_BLOCKS,),
        in_specs=[
            pl.BlockSpec((BLK, ATOM_DIM), lambda i: (i, 0)),
            pl.BlockSpec((BLK, K_AUG), lambda i: (i, 0)),
            pl.BlockSpec((K_AUG, M_DIM), lambda i: (0, 0)),
            pl.BlockSpec((ATOM_DIM, M_DIM), lambda i: (0, 0)),
            pl.BlockSpec((M_DIM, ATOM_DIM), lambda i: (0, 0)),
        ],
        out_specs=pl.BlockSpec((BLK, ATOM_DIM), lambda i: (i, 0)),
        out_shape=jax.ShapeDtypeStruct((E_PAD, ATOM_DIM), jnp.float32),
        compiler_params=pltpu.CompilerParams(
            dimension_semantics=("arbitrary",)
        ),
    )(nb, bond_aug, rmat, tmat, kperm)

    out_pad = _scatter_add(
        src_pad.reshape(NT, OUTER, SCHUNKS, CHUNK),
        transformed.reshape(NT, OUTER, SCHUNKS, CHUNK, ATOM_DIM),
        jnp.zeros((ZCHUNK, ATOM_DIM), jnp.float32),
    )
    return out_pad[:N_NODES]


# speedup vs baseline: 3.4155x; 1.5058x over previous
import functools

import jax
import jax.numpy as jnp
from jax import lax
from jax.experimental import pallas as pl
from jax.experimental.pallas import tpu as pltpu
from jax.experimental.pallas import tpu_sc as plsc

N_NODES = 50000
ATOM_DIM = 32
BOND_DIM = 16
M_DIM = BOND_DIM * ATOM_DIM

NC = 2
NT = 16
NW = NC * NT
CHUNK = 128

CPW = 25
EDGES_PW = CPW * CHUNK
E_PAD = NW * EDGES_PW

BLK = 1024

HALF = 25088
OUT_PAD = 2 * HALF
DRAIN_PT = HALF // NT
ACC_ROWS = 26624
ZCHUNK = ACC_ROWS // NT
DUMMY = HALF
EPT = E_PAD // NT
SCHUNKS = 10
SSTEP = SCHUNKS * CHUNK
OUTER = EPT // SSTEP
SENTINEL = 1 << 30

_mesh = plsc.VectorSubcoreMesh(core_axis_name="c", subcore_axis_name="s")


@functools.partial(
    pl.kernel,
    out_type=jax.ShapeDtypeStruct((E_PAD, ATOM_DIM), jnp.float32),
    mesh=_mesh,
    scratch_types=[
        pltpu.VMEM((EDGES_PW,), jnp.int32),
        pltpu.VMEM((CPW, CHUNK, ATOM_DIM), jnp.float32),
        pltpu.SemaphoreType.DMA,
        pltpu.SemaphoreType.DMA,
    ],
    compiler_params=pltpu.CompilerParams(use_tc_tiling_on_sc=False),
)
def _gather_rows(dst_hbm, atom_hbm, out_hbm, idx_v, rows_v, gsem, osem):
    wid = lax.axis_index("s") * NC + lax.axis_index("c")
    base = wid * EDGES_PW
    pltpu.sync_copy(dst_hbm.at[pl.ds(base, EDGES_PW)], idx_v)
    gathers = [
        pltpu.async_copy(
            atom_hbm.at[idx_v.at[pl.ds(j * CHUNK, CHUNK)]], rows_v.at[j], gsem
        )
        for j in range(CPW)
    ]
    stores = []
    for j in range(CPW):
        gathers[j].wait()
        stores.append(
            pltpu.async_copy(
                rows_v.at[j], out_hbm.at[pl.ds(base + j * CHUNK, CHUNK)], osem
            )
        )
    for d in stores:
        d.wait()


def _dense_body(nb_ref, bond_ref, rmat_ref, tmat_ref, kperm_ref, kbias_ref, out_ref):
    a = jnp.dot(bond_ref[...], rmat_ref[...], preferred_element_type=jnp.float32)
    b = jnp.dot(nb_ref[...], tmat_ref[...], preferred_element_type=jnp.float32)
    out_ref[...] = jnp.dot(
        a * b, kperm_ref[...], preferred_element_type=jnp.float32
    ) + jnp.dot(nb_ref[...], kbias_ref[...], preferred_element_type=jnp.float32)


@functools.partial(
    pl.kernel,
    out_type=jax.ShapeDtypeStruct((OUT_PAD, ATOM_DIM), jnp.float32),
    mesh=_mesh,
    scratch_types=[
        pltpu.VMEM((SSTEP,), jnp.int32),
        pltpu.VMEM((SCHUNKS, CHUNK), jnp.int32),
        pltpu.VMEM((SSTEP, ATOM_DIM), jnp.float32),
        pltpu.VMEM_SHARED((ACC_ROWS, ATOM_DIM), jnp.float32),
    ],
    compiler_params=pltpu.CompilerParams(use_tc_tiling_on_sc=False),
)
def _scatter_add(src_hbm, t_hbm, zero_hbm, out_hbm, sidx_v, lidx_v, tv, acc):
    c = lax.axis_index("c")
    s = lax.axis_index("s")
    pltpu.sync_copy(zero_hbm, acc.at[pl.ds(s * ZCHUNK, ZCHUNK)])
    plsc.subcore_barrier()
    base = c * HALF

    def outer(o, carry):
        ebase = s * EPT + o * SSTEP
        pltpu.sync_copy(src_hbm.at[pl.ds(ebase, SSTEP)], sidx_v)
        pltpu.sync_copy(t_hbm.at[pl.ds(ebase, SSTEP)], tv)
        for r in range(SCHUNKS):
            for q in range(CHUNK // 16):
                v = sidx_v[pl.ds(r * CHUNK + q * 16, 16)]
                il = v - base
                ok = (il >= 0) & (il < HALF)
                lidx_v[r, pl.ds(q * 16, 16)] = jnp.where(ok, il, DUMMY)
        for r in range(SCHUNKS):
            pltpu.sync_copy(
                tv.at[pl.ds(r * CHUNK, CHUNK)], acc.at[lidx_v.at[r]], add=True
            )
        return carry

    lax.fori_loop(0, OUTER, outer, 0)
    plsc.subcore_barrier()
    pltpu.sync_copy(
        acc.at[pl.ds(s * DRAIN_PT, DRAIN_PT)],
        out_hbm.at[pl.ds(c * HALF + s * DRAIN_PT, DRAIN_PT)],
    )


def kernel(atom_features, bond_features, pair_indices, kernel, bias):
    pi = pair_indices.astype(jnp.int32)
    src, dst = pi[:, 0], pi[:, 1]
    n_edges = src.shape[0]
    pad = E_PAD - n_edges

    dst_pad = jnp.concatenate([dst, jnp.zeros((pad,), jnp.int32)])
    src_pad = jnp.concatenate([src, jnp.full((pad,), SENTINEL, jnp.int32)])

    kperm = kernel.reshape(BOND_DIM, ATOM_DIM, ATOM_DIM).transpose(0, 2, 1).reshape(
        M_DIM, ATOM_DIM
    )
    kbias = bias.reshape(ATOM_DIM, ATOM_DIM).T
    rmat = jnp.repeat(jnp.eye(BOND_DIM, dtype=jnp.float32), ATOM_DIM, axis=1)
    tmat = jnp.tile(jnp.eye(ATOM_DIM, dtype=jnp.float32), (1, BOND_DIM))

    nb = _gather_rows(dst_pad, atom_features)

    n_blocks = (n_edges + BLK - 1) // BLK
    transformed = pl.pallas_call(
        _dense_body,
        grid=(n_blocks,),
        in_specs=[
            pl.BlockSpec((BLK, ATOM_DIM), lambda i: (i, 0)),
            pl.BlockSpec((BLK, BOND_DIM), lambda i: (i, 0)),
            pl.BlockSpec((BOND_DIM, M_DIM), lambda i: (0, 0)),
            pl.BlockSpec((ATOM_DIM, M_DIM), lambda i: (0, 0)),
            pl.BlockSpec((M_DIM, ATOM_DIM), lambda i: (0, 0)),
            pl.BlockSpec((ATOM_DIM, ATOM_DIM), lambda i: (0, 0)),
        ],
        out_specs=pl.BlockSpec((BLK, ATOM_DIM), lambda i: (i, 0)),
        out_shape=jax.ShapeDtypeStruct((E_PAD, ATOM_DIM), jnp.float32),
        compiler_params=pltpu.CompilerParams(
            dimension_semantics=("arbitrary",)
        ),
    )(nb, bond_features, rmat, tmat, kperm, kbias)

    out_pad = _scatter_add(
        src_pad,
        transformed,
        jnp.zeros((ZCHUNK, ATOM_DIM), jnp.float32),
    )
    return out_pad[:N_NODES]


# speedup vs baseline: 4.0865x; 1.1964x over previous
import functools

import jax
import jax.numpy as jnp
from jax import lax
from jax.experimental import pallas as pl
from jax.experimental.pallas import tpu as pltpu
from jax.experimental.pallas import tpu_sc as plsc

N_NODES = 50000
ATOM_DIM = 32
BOND_DIM = 16
M_DIM = BOND_DIM * ATOM_DIM
PACK = 128 // ATOM_DIM

NC = 2
NT = 16
NW = NC * NT
CHUNK = 128

CPW = 25
EDGES_PW = CPW * CHUNK
E_PAD = NW * EDGES_PW
EP4 = E_PAD // PACK

BLK = 1024

HALF = 25088
OUT_PAD = 2 * HALF
DRAIN_PT = HALF // NT
ACC_ROWS = 26624
ZCHUNK = ACC_ROWS // NT
DUMMY = HALF
EPT = E_PAD // NT
SCHUNKS = 10
SSTEP = SCHUNKS * CHUNK
OUTER = EPT // SSTEP
SENTINEL = 1 << 30

_mesh = plsc.VectorSubcoreMesh(core_axis_name="c", subcore_axis_name="s")


@functools.partial(
    pl.kernel,
    out_type=jax.ShapeDtypeStruct((E_PAD, 128), jnp.float32),
    mesh=_mesh,
    scratch_types=[
        pltpu.VMEM((EDGES_PW,), jnp.int32),
        pltpu.VMEM((CPW, CHUNK, ATOM_DIM), jnp.float32),
        pltpu.SemaphoreType.DMA,
        pltpu.SemaphoreType.DMA,
    ],
    compiler_params=pltpu.CompilerParams(use_tc_tiling_on_sc=False),
)
def _gather_rows(dst_hbm, atom_hbm, out_hbm, idx_v, rows_v, gsem, osem):
    wid = lax.axis_index("s") * NC + lax.axis_index("c")
    base = wid * EDGES_PW
    pltpu.sync_copy(dst_hbm.at[pl.ds(base, EDGES_PW)], idx_v)
    gathers = [
        pltpu.async_copy(
            atom_hbm.at[idx_v.at[pl.ds(j * CHUNK, CHUNK)]], rows_v.at[j], gsem
        )
        for j in range(CPW)
    ]
    stores = []
    for j in range(CPW):
        gathers[j].wait()
        stores.append(
            pltpu.async_copy(
                rows_v.at[j],
                out_hbm.at[pl.ds(base + j * CHUNK, CHUNK), pl.ds(0, ATOM_DIM)],
                osem,
            )
        )
    for d in stores:
        d.wait()


def _dense_body(nbp_ref, bond_ref, rmat_ref, tmat_ref, kperm_ref, kbias_ref, out_ref):
    nb = nbp_ref[...][:, :ATOM_DIM]
    a = jnp.dot(bond_ref[...], rmat_ref[...], preferred_element_type=jnp.float32)
    b = jnp.dot(nb, tmat_ref[...], preferred_element_type=jnp.float32)
    out_ref[:, :ATOM_DIM] = jnp.dot(
        a * b, kperm_ref[...], preferred_element_type=jnp.float32
    ) + jnp.dot(nb, kbias_ref[...], preferred_element_type=jnp.float32)


@functools.partial(
    pl.kernel,
    out_type=jax.ShapeDtypeStruct((OUT_PAD, ATOM_DIM), jnp.float32),
    mesh=_mesh,
    scratch_types=[
        pltpu.VMEM((SSTEP,), jnp.int32),
        pltpu.VMEM((SCHUNKS, CHUNK), jnp.int32),
        pltpu.VMEM((SSTEP, ATOM_DIM), jnp.float32),
        pltpu.VMEM_SHARED((ACC_ROWS, ATOM_DIM), jnp.float32),
    ],
    compiler_params=pltpu.CompilerParams(use_tc_tiling_on_sc=False),
)
def _scatter_add(src_hbm, t_hbm, zero_hbm, out_hbm, sidx_v, lidx_v, tv, acc):
    c = lax.axis_index("c")
    s = lax.axis_index("s")
    pltpu.sync_copy(zero_hbm, acc.at[pl.ds(s * ZCHUNK, ZCHUNK)])
    plsc.subcore_barrier()
    base = c * HALF

    def outer(o, carry):
        ebase = s * EPT + o * SSTEP
        pltpu.sync_copy(src_hbm.at[pl.ds(ebase, SSTEP)], sidx_v)
        pltpu.sync_copy(
            t_hbm.at[pl.ds(ebase, SSTEP), pl.ds(0, ATOM_DIM)], tv
        )
        for r in range(SCHUNKS):
            for q in range(CHUNK // 16):
                v = sidx_v[pl.ds(r * CHUNK + q * 16, 16)]
                il = v - base
                ok = (il >= 0) & (il < HALF)
                lidx_v[r, pl.ds(q * 16, 16)] = jnp.where(ok, il, DUMMY)
        for r in range(SCHUNKS):
            pltpu.sync_copy(
                tv.at[pl.ds(r * CHUNK, CHUNK)], acc.at[lidx_v.at[r]], add=True
            )
        return carry

    lax.fori_loop(0, OUTER, outer, 0)
    plsc.subcore_barrier()
    pltpu.sync_copy(
        acc.at[pl.ds(s * DRAIN_PT, DRAIN_PT)],
        out_hbm.at[pl.ds(c * HALF + s * DRAIN_PT, DRAIN_PT)],
    )


def kernel(atom_features, bond_features, pair_indices, kernel, bias):
    pi = pair_indices.astype(jnp.int32)
    src, dst = pi[:, 0], pi[:, 1]
    n_edges = src.shape[0]
    pad = E_PAD - n_edges

    dst_pad = jnp.concatenate([dst, jnp.zeros((pad,), jnp.int32)])
    src_pad = jnp.concatenate([src, jnp.full((pad,), SENTINEL, jnp.int32)])

    kperm = kernel.reshape(BOND_DIM, ATOM_DIM, ATOM_DIM).transpose(0, 2, 1).reshape(
        M_DIM, ATOM_DIM
    )
    kbias = bias.reshape(ATOM_DIM, ATOM_DIM).T
    rmat = jnp.repeat(jnp.eye(BOND_DIM, dtype=jnp.float32), ATOM_DIM, axis=1)
    tmat = jnp.tile(jnp.eye(ATOM_DIM, dtype=jnp.float32), (1, BOND_DIM))

    nb_packed = _gather_rows(dst_pad, atom_features)

    n_blocks = (n_edges + BLK - 1) // BLK
    transformed = pl.pallas_call(
        _dense_body,
        grid=(n_blocks,),
        in_specs=[
            pl.BlockSpec((BLK, 128), lambda i: (i, 0)),
            pl.BlockSpec((BLK, BOND_DIM), lambda i: (i, 0)),
            pl.BlockSpec((BOND_DIM, M_DIM), lambda i: (0, 0)),
            pl.BlockSpec((ATOM_DIM, M_DIM), lambda i: (0, 0)),
            pl.BlockSpec((M_DIM, ATOM_DIM), lambda i: (0, 0)),
            pl.BlockSpec((ATOM_DIM, ATOM_DIM), lambda i: (0, 0)),
        ],
        out_specs=pl.BlockSpec((BLK, 128), lambda i: (i, 0)),
        out_shape=jax.ShapeDtypeStruct((E_PAD, 128), jnp.float32),
        compiler_params=pltpu.CompilerParams(
            dimension_semantics=("arbitrary",)
        ),
    )(nb_packed, bond_features, rmat, tmat, kperm, kbias)

    out_pad = _scatter_add(
        src_pad,
        transformed,
        jnp.zeros((ZCHUNK, ATOM_DIM), jnp.float32),
    )
    return out_pad[:N_NODES]


# speedup vs baseline: 4.0891x; 1.0006x over previous
import functools

import jax
import jax.numpy as jnp
from jax import lax
from jax.experimental import pallas as pl
from jax.experimental.pallas import tpu as pltpu
from jax.experimental.pallas import tpu_sc as plsc

N_NODES = 50000
ATOM_DIM = 32
BOND_DIM = 16
M_DIM = BOND_DIM * ATOM_DIM
PACK = 128 // ATOM_DIM

NC = 2
NT = 16
NW = NC * NT
CHUNK = 128

CPW = 25
EDGES_PW = CPW * CHUNK
E_PAD = NW * EDGES_PW
EP4 = E_PAD // PACK

BLK = 1024

HALF = 25088
OUT_PAD = 2 * HALF
DRAIN_PT = HALF // NT
ACC_ROWS = 26624
ZCHUNK = ACC_ROWS // NT
DUMMY = HALF
EPT = E_PAD // NT
SCHUNKS = 10
SSTEP = SCHUNKS * CHUNK
OUTER = EPT // SSTEP
SENTINEL = 1 << 30

_mesh = plsc.VectorSubcoreMesh(core_axis_name="c", subcore_axis_name="s")


@functools.partial(
    pl.kernel,
    out_type=jax.ShapeDtypeStruct((E_PAD, 128), jnp.float32),
    mesh=_mesh,
    scratch_types=[
        pltpu.VMEM((EDGES_PW,), jnp.int32),
        pltpu.VMEM((CPW, CHUNK, ATOM_DIM), jnp.float32),
        pltpu.SemaphoreType.DMA,
        pltpu.SemaphoreType.DMA,
    ],
    compiler_params=pltpu.CompilerParams(use_tc_tiling_on_sc=False),
)
def _gather_rows(dst_hbm, atom_hbm, out_hbm, idx_v, rows_v, gsem, osem):
    wid = lax.axis_index("s") * NC + lax.axis_index("c")
    base = wid * EDGES_PW
    pltpu.sync_copy(dst_hbm.at[pl.ds(base, EDGES_PW)], idx_v)
    gathers = [
        pltpu.async_copy(
            atom_hbm.at[idx_v.at[pl.ds(j * CHUNK, CHUNK)]], rows_v.at[j], gsem
        )
        for j in range(CPW)
    ]
    stores = []
    for j in range(CPW):
        gathers[j].wait()
        stores.append(
            pltpu.async_copy(
                rows_v.at[j],
                out_hbm.at[pl.ds(base + j * CHUNK, CHUNK), pl.ds(0, ATOM_DIM)],
                osem,
            )
        )
    for d in stores:
        d.wait()


def _dense_body(nbp_ref, bond_ref, rmat_ref, tmat_ref, kperm_ref, kbias_ref, out_ref):
    nb = nbp_ref[...][:, :ATOM_DIM].astype(jnp.bfloat16)
    a = jnp.dot(
        bond_ref[...].astype(jnp.bfloat16),
        rmat_ref[...],
        preferred_element_type=jnp.float32,
    ).astype(jnp.bfloat16)
    b = jnp.dot(
        nb, tmat_ref[...], preferred_element_type=jnp.float32
    ).astype(jnp.bfloat16)
    out_ref[:, :ATOM_DIM] = jnp.dot(
        a * b, kperm_ref[...], preferred_element_type=jnp.float32
    ) + jnp.dot(nb, kbias_ref[...], preferred_element_type=jnp.float32)


@functools.partial(
    pl.kernel,
    out_type=jax.ShapeDtypeStruct((OUT_PAD, ATOM_DIM), jnp.float32),
    mesh=_mesh,
    scratch_types=[
        pltpu.VMEM((SSTEP,), jnp.int32),
        pltpu.VMEM((SCHUNKS, CHUNK), jnp.int32),
        pltpu.VMEM((SSTEP, ATOM_DIM), jnp.float32),
        pltpu.VMEM_SHARED((ACC_ROWS, ATOM_DIM), jnp.float32),
    ],
    compiler_params=pltpu.CompilerParams(use_tc_tiling_on_sc=False),
)
def _scatter_add(src_hbm, t_hbm, zero_hbm, out_hbm, sidx_v, lidx_v, tv, acc):
    c = lax.axis_index("c")
    s = lax.axis_index("s")
    pltpu.sync_copy(zero_hbm, acc.at[pl.ds(s * ZCHUNK, ZCHUNK)])
    plsc.subcore_barrier()
    base = c * HALF

    def outer(o, carry):
        ebase = s * EPT + o * SSTEP
        pltpu.sync_copy(src_hbm.at[pl.ds(ebase, SSTEP)], sidx_v)
        pltpu.sync_copy(
            t_hbm.at[pl.ds(ebase, SSTEP), pl.ds(0, ATOM_DIM)], tv
        )
        for r in range(SCHUNKS):
            for q in range(CHUNK // 16):
                v = sidx_v[pl.ds(r * CHUNK + q * 16, 16)]
                il = v - base
                ok = (il >= 0) & (il < HALF)
                lidx_v[r, pl.ds(q * 16, 16)] = jnp.where(ok, il, DUMMY)
        for r in range(SCHUNKS):
            pltpu.sync_copy(
                tv.at[pl.ds(r * CHUNK, CHUNK)], acc.at[lidx_v.at[r]], add=True
            )
        return carry

    lax.fori_loop(0, OUTER, outer, 0)
    plsc.subcore_barrier()
    pltpu.sync_copy(
        acc.at[pl.ds(s * DRAIN_PT, DRAIN_PT)],
        out_hbm.at[pl.ds(c * HALF + s * DRAIN_PT, DRAIN_PT)],
    )


def kernel(atom_features, bond_features, pair_indices, kernel, bias):
    pi = pair_indices.astype(jnp.int32)
    src, dst = pi[:, 0], pi[:, 1]
    n_edges = src.shape[0]
    pad = E_PAD - n_edges

    dst_pad = jnp.concatenate([dst, jnp.zeros((pad,), jnp.int32)])
    src_pad = jnp.concatenate([src, jnp.full((pad,), SENTINEL, jnp.int32)])

    kperm = kernel.reshape(BOND_DIM, ATOM_DIM, ATOM_DIM).transpose(0, 2, 1).reshape(
        M_DIM, ATOM_DIM
    )
    kbias = bias.reshape(ATOM_DIM, ATOM_DIM).T
    kperm = kperm.astype(jnp.bfloat16)
    rmat = jnp.repeat(jnp.eye(BOND_DIM, dtype=jnp.bfloat16), ATOM_DIM, axis=1)
    tmat = jnp.tile(jnp.eye(ATOM_DIM, dtype=jnp.bfloat16), (1, BOND_DIM))

    nb_packed = _gather_rows(dst_pad, atom_features)

    n_blocks = (n_edges + BLK - 1) // BLK
    transformed = pl.pallas_call(
        _dense_body,
        grid=(n_blocks,),
        in_specs=[
            pl.BlockSpec((BLK, 128), lambda i: (i, 0)),
            pl.BlockSpec((BLK, BOND_DIM), lambda i: (i, 0)),
            pl.BlockSpec((BOND_DIM, M_DIM), lambda i: (0, 0)),
            pl.BlockSpec((ATOM_DIM, M_DIM), lambda i: (0, 0)),
            pl.BlockSpec((M_DIM, ATOM_DIM), lambda i: (0, 0)),
            pl.BlockSpec((ATOM_DIM, ATOM_DIM), lambda i: (0, 0)),
        ],
        out_specs=pl.BlockSpec((BLK, 128), lambda i: (i, 0)),
        out_shape=jax.ShapeDtypeStruct((E_PAD, 128), jnp.float32),
        compiler_params=pltpu.CompilerParams(
            dimension_semantics=("arbitrary",)
        ),
    )(nb_packed, bond_features, rmat, tmat, kperm, kbias)

    out_pad = _scatter_add(
        src_pad,
        transformed,
        jnp.zeros((ZCHUNK, ATOM_DIM), jnp.float32),
    )
    return out_pad[:N_NODES]


# speedup vs baseline: 4.4075x; 1.0779x over previous
import functools

import jax
import jax.numpy as jnp
from jax import lax
from jax.experimental import pallas as pl
from jax.experimental.pallas import tpu as pltpu
from jax.experimental.pallas import tpu_sc as plsc

N_NODES = 50000
ATOM_DIM = 32
BOND_DIM = 16
M_DIM = BOND_DIM * ATOM_DIM
PACK = 128 // ATOM_DIM

NC = 2
NT = 16
NW = NC * NT
CHUNK = 128

CPW = 25
EDGES_PW = CPW * CHUNK
E_PAD = NW * EDGES_PW
EP4 = E_PAD // PACK

BLK = 2048

HALF = 25088
OUT_PAD = 2 * HALF
DRAIN_PT = HALF // NT
ACC_ROWS = 26624
ZCHUNK = ACC_ROWS // NT
DUMMY = HALF
EPT = E_PAD // NT
SCHUNKS = 10
SSTEP = SCHUNKS * CHUNK
OUTER = EPT // SSTEP
SENTINEL = 1 << 30

_mesh = plsc.VectorSubcoreMesh(core_axis_name="c", subcore_axis_name="s")


@functools.partial(
    pl.kernel,
    out_type=jax.ShapeDtypeStruct((E_PAD, 128), jnp.float32),
    mesh=_mesh,
    scratch_types=[
        pltpu.VMEM((EDGES_PW,), jnp.int32),
        pltpu.VMEM((CPW, CHUNK, ATOM_DIM), jnp.float32),
        pltpu.SemaphoreType.DMA,
        pltpu.SemaphoreType.DMA,
    ],
    compiler_params=pltpu.CompilerParams(use_tc_tiling_on_sc=False),
)
def _gather_rows(dst_hbm, atom_hbm, out_hbm, idx_v, rows_v, gsem, osem):
    wid = lax.axis_index("s") * NC + lax.axis_index("c")
    base = wid * EDGES_PW
    pltpu.sync_copy(dst_hbm.at[pl.ds(base, EDGES_PW)], idx_v)
    gathers = [
        pltpu.async_copy(
            atom_hbm.at[idx_v.at[pl.ds(j * CHUNK, CHUNK)]], rows_v.at[j], gsem
        )
        for j in range(CPW)
    ]
    stores = []
    for j in range(CPW):
        gathers[j].wait()
        stores.append(
            pltpu.async_copy(
                rows_v.at[j],
                out_hbm.at[pl.ds(base + j * CHUNK, CHUNK), pl.ds(0, ATOM_DIM)],
                osem,
            )
        )
    for d in stores:
        d.wait()


def _dense_body(nbp_ref, bond_ref, rmat_ref, tmat_ref, kperm_ref, kbias_ref, out_ref):
    nb = nbp_ref[...][:, :ATOM_DIM].astype(jnp.bfloat16)
    a = jnp.dot(
        bond_ref[...].astype(jnp.bfloat16),
        rmat_ref[...],
        preferred_element_type=jnp.float32,
    ).astype(jnp.bfloat16)
    b = jnp.dot(
        nb, tmat_ref[...], preferred_element_type=jnp.float32
    ).astype(jnp.bfloat16)
    out_ref[:, :ATOM_DIM] = jnp.dot(
        a * b, kperm_ref[...], preferred_element_type=jnp.float32
    ) + jnp.dot(nb, kbias_ref[...], preferred_element_type=jnp.float32)


@functools.partial(
    pl.kernel,
    out_type=jax.ShapeDtypeStruct((OUT_PAD, ATOM_DIM), jnp.float32),
    mesh=_mesh,
    scratch_types=[
        pltpu.VMEM((SSTEP,), jnp.int32),
        pltpu.VMEM((SCHUNKS, CHUNK), jnp.int32),
        pltpu.VMEM((SSTEP, ATOM_DIM), jnp.float32),
        pltpu.VMEM_SHARED((ACC_ROWS, ATOM_DIM), jnp.float32),
    ],
    compiler_params=pltpu.CompilerParams(use_tc_tiling_on_sc=False),
)
def _scatter_add(src_hbm, t_hbm, zero_hbm, out_hbm, sidx_v, lidx_v, tv, acc):
    c = lax.axis_index("c")
    s = lax.axis_index("s")
    pltpu.sync_copy(zero_hbm, acc.at[pl.ds(s * ZCHUNK, ZCHUNK)])
    plsc.subcore_barrier()
    base = c * HALF

    def outer(o, carry):
        ebase = s * EPT + o * SSTEP
        pltpu.sync_copy(src_hbm.at[pl.ds(ebase, SSTEP)], sidx_v)
        pltpu.sync_copy(
            t_hbm.at[pl.ds(ebase, SSTEP), pl.ds(0, ATOM_DIM)], tv
        )
        for r in range(SCHUNKS):
            for q in range(CHUNK // 16):
                v = sidx_v[pl.ds(r * CHUNK + q * 16, 16)]
                il = v - base
                ok = (il >= 0) & (il < HALF)
                lidx_v[r, pl.ds(q * 16, 16)] = jnp.where(ok, il, DUMMY)
        for r in range(SCHUNKS):
            pltpu.sync_copy(
                tv.at[pl.ds(r * CHUNK, CHUNK)], acc.at[lidx_v.at[r]], add=True
            )
        return carry

    lax.fori_loop(0, OUTER, outer, 0)
    plsc.subcore_barrier()
    pltpu.sync_copy(
        acc.at[pl.ds(s * DRAIN_PT, DRAIN_PT)],
        out_hbm.at[pl.ds(c * HALF + s * DRAIN_PT, DRAIN_PT)],
    )


def kernel(atom_features, bond_features, pair_indices, kernel, bias):
    pi = pair_indices.astype(jnp.int32)
    src, dst = pi[:, 0], pi[:, 1]
    n_edges = src.shape[0]
    pad = E_PAD - n_edges

    dst_pad = jnp.concatenate([dst, jnp.zeros((pad,), jnp.int32)])
    src_pad = jnp.concatenate([src, jnp.full((pad,), SENTINEL, jnp.int32)])

    kperm = kernel.reshape(BOND_DIM, ATOM_DIM, ATOM_DIM).transpose(0, 2, 1).reshape(
        M_DIM, ATOM_DIM
    )
    kbias = bias.reshape(ATOM_DIM, ATOM_DIM).T
    kperm = kperm.astype(jnp.bfloat16)
    rmat = jnp.repeat(jnp.eye(BOND_DIM, dtype=jnp.bfloat16), ATOM_DIM, axis=1)
    tmat = jnp.tile(jnp.eye(ATOM_DIM, dtype=jnp.bfloat16), (1, BOND_DIM))

    nb_packed = _gather_rows(dst_pad, atom_features)

    n_blocks = (n_edges + BLK - 1) // BLK
    transformed = pl.pallas_call(
        _dense_body,
        grid=(n_blocks,),
        in_specs=[
            pl.BlockSpec((BLK, 128), lambda i: (i, 0)),
            pl.BlockSpec((BLK, BOND_DIM), lambda i: (i, 0)),
            pl.BlockSpec((BOND_DIM, M_DIM), lambda i: (0, 0)),
            pl.BlockSpec((ATOM_DIM, M_DIM), lambda i: (0, 0)),
            pl.BlockSpec((M_DIM, ATOM_DIM), lambda i: (0, 0)),
            pl.BlockSpec((ATOM_DIM, ATOM_DIM), lambda i: (0, 0)),
        ],
        out_specs=pl.BlockSpec((BLK, 128), lambda i: (i, 0)),
        out_shape=jax.ShapeDtypeStruct((E_PAD, 128), jnp.float32),
        compiler_params=pltpu.CompilerParams(
            dimension_semantics=("arbitrary",)
        ),
    )(nb_packed, bond_features, rmat, tmat, kperm, kbias)

    out_pad = _scatter_add(
        src_pad,
        transformed,
        jnp.zeros((ZCHUNK, ATOM_DIM), jnp.float32),
    )
    return out_pad[:N_NODES]


# speedup vs baseline: 4.4456x; 1.0087x over previous
import functools

import jax
import jax.numpy as jnp
from jax import lax
from jax.experimental import pallas as pl
from jax.experimental.pallas import tpu as pltpu
from jax.experimental.pallas import tpu_sc as plsc

N_NODES = 50000
ATOM_DIM = 32
BOND_DIM = 16
M_DIM = BOND_DIM * ATOM_DIM
PACK = 128 // ATOM_DIM

NC = 2
NT = 16
NW = NC * NT
CHUNK = 128

CPW = 25
EDGES_PW = CPW * CHUNK
E_PAD = NW * EDGES_PW
EP4 = E_PAD // PACK

BLK = 4096

HALF = 25088
OUT_PAD = 2 * HALF
DRAIN_PT = HALF // NT
ACC_ROWS = 26624
ZCHUNK = ACC_ROWS // NT
DUMMY = HALF
EPT = E_PAD // NT
SCHUNKS = 10
SSTEP = SCHUNKS * CHUNK
OUTER = EPT // SSTEP
SENTINEL = 1 << 30

_mesh = plsc.VectorSubcoreMesh(core_axis_name="c", subcore_axis_name="s")


@functools.partial(
    pl.kernel,
    out_type=jax.ShapeDtypeStruct((E_PAD, 128), jnp.float32),
    mesh=_mesh,
    scratch_types=[
        pltpu.VMEM((EDGES_PW,), jnp.int32),
        pltpu.VMEM((CPW, CHUNK, ATOM_DIM), jnp.float32),
        pltpu.SemaphoreType.DMA,
        pltpu.SemaphoreType.DMA,
    ],
    compiler_params=pltpu.CompilerParams(use_tc_tiling_on_sc=False),
)
def _gather_rows(dst_hbm, atom_hbm, out_hbm, idx_v, rows_v, gsem, osem):
    wid = lax.axis_index("s") * NC + lax.axis_index("c")
    base = wid * EDGES_PW
    pltpu.sync_copy(dst_hbm.at[pl.ds(base, EDGES_PW)], idx_v)
    gathers = [
        pltpu.async_copy(
            atom_hbm.at[idx_v.at[pl.ds(j * CHUNK, CHUNK)]], rows_v.at[j], gsem
        )
        for j in range(CPW)
    ]
    stores = []
    for j in range(CPW):
        gathers[j].wait()
        stores.append(
            pltpu.async_copy(
                rows_v.at[j],
                out_hbm.at[pl.ds(base + j * CHUNK, CHUNK), pl.ds(0, ATOM_DIM)],
                osem,
            )
        )
    for d in stores:
        d.wait()


def _dense_body(nbp_ref, bond_ref, rmat_ref, tmat_ref, kperm_ref, kbias_ref, out_ref):
    nb = nbp_ref[...][:, :ATOM_DIM].astype(jnp.bfloat16)
    a = jnp.dot(
        bond_ref[...].astype(jnp.bfloat16),
        rmat_ref[...],
        preferred_element_type=jnp.float32,
    ).astype(jnp.bfloat16)
    b = jnp.dot(
        nb, tmat_ref[...], preferred_element_type=jnp.float32
    ).astype(jnp.bfloat16)
    out_ref[:, :ATOM_DIM] = jnp.dot(
        a * b, kperm_ref[...], preferred_element_type=jnp.float32
    ) + jnp.dot(nb, kbias_ref[...], preferred_element_type=jnp.float32)


@functools.partial(
    pl.kernel,
    out_type=jax.ShapeDtypeStruct((OUT_PAD, ATOM_DIM), jnp.float32),
    mesh=_mesh,
    scratch_types=[
        pltpu.VMEM((SSTEP,), jnp.int32),
        pltpu.VMEM((SCHUNKS, CHUNK), jnp.int32),
        pltpu.VMEM((SSTEP, ATOM_DIM), jnp.float32),
        pltpu.VMEM_SHARED((ACC_ROWS, ATOM_DIM), jnp.float32),
    ],
    compiler_params=pltpu.CompilerParams(use_tc_tiling_on_sc=False),
)
def _scatter_add(src_hbm, t_hbm, zero_hbm, out_hbm, sidx_v, lidx_v, tv, acc):
    c = lax.axis_index("c")
    s = lax.axis_index("s")
    pltpu.sync_copy(zero_hbm, acc.at[pl.ds(s * ZCHUNK, ZCHUNK)])
    plsc.subcore_barrier()
    base = c * HALF

    def outer(o, carry):
        ebase = s * EPT + o * SSTEP
        pltpu.sync_copy(src_hbm.at[pl.ds(ebase, SSTEP)], sidx_v)
        pltpu.sync_copy(
            t_hbm.at[pl.ds(ebase, SSTEP), pl.ds(0, ATOM_DIM)], tv
        )
        for r in range(SCHUNKS):
            for q in range(CHUNK // 16):
                v = sidx_v[pl.ds(r * CHUNK + q * 16, 16)]
                il = v - base
                ok = (il >= 0) & (il < HALF)
                lidx_v[r, pl.ds(q * 16, 16)] = jnp.where(ok, il, DUMMY)
        for r in range(SCHUNKS):
            pltpu.sync_copy(
                tv.at[pl.ds(r * CHUNK, CHUNK)], acc.at[lidx_v.at[r]], add=True
            )
        return carry

    lax.fori_loop(0, OUTER, outer, 0)
    plsc.subcore_barrier()
    pltpu.sync_copy(
        acc.at[pl.ds(s * DRAIN_PT, DRAIN_PT)],
        out_hbm.at[pl.ds(c * HALF + s * DRAIN_PT, DRAIN_PT)],
    )


def kernel(atom_features, bond_features, pair_indices, kernel, bias):
    pi = pair_indices.astype(jnp.int32)
    src, dst = pi[:, 0], pi[:, 1]
    n_edges = src.shape[0]
    pad = E_PAD - n_edges

    dst_pad = jnp.concatenate([dst, jnp.zeros((pad,), jnp.int32)])
    src_pad = jnp.concatenate([src, jnp.full((pad,), SENTINEL, jnp.int32)])

    kperm = kernel.reshape(BOND_DIM, ATOM_DIM, ATOM_DIM).transpose(0, 2, 1).reshape(
        M_DIM, ATOM_DIM
    )
    kbias = bias.reshape(ATOM_DIM, ATOM_DIM).T
    kperm = kperm.astype(jnp.bfloat16)
    rmat = jnp.repeat(jnp.eye(BOND_DIM, dtype=jnp.bfloat16), ATOM_DIM, axis=1)
    tmat = jnp.tile(jnp.eye(ATOM_DIM, dtype=jnp.bfloat16), (1, BOND_DIM))

    nb_packed = _gather_rows(dst_pad, atom_features)

    n_blocks = (n_edges + BLK - 1) // BLK
    transformed = pl.pallas_call(
        _dense_body,
        grid=(n_blocks,),
        in_specs=[
            pl.BlockSpec((BLK, 128), lambda i: (i, 0)),
            pl.BlockSpec((BLK, BOND_DIM), lambda i: (i, 0)),
            pl.BlockSpec((BOND_DIM, M_DIM), lambda i: (0, 0)),
            pl.BlockSpec((ATOM_DIM, M_DIM), lambda i: (0, 0)),
            pl.BlockSpec((M_DIM, ATOM_DIM), lambda i: (0, 0)),
            pl.BlockSpec((ATOM_DIM, ATOM_DIM), lambda i: (0, 0)),
        ],
        out_specs=pl.BlockSpec((BLK, 128), lambda i: (i, 0)),
        out_shape=jax.ShapeDtypeStruct((E_PAD, 128), jnp.float32),
        compiler_params=pltpu.CompilerParams(
            dimension_semantics=("arbitrary",)
        ),
    )(nb_packed, bond_features, rmat, tmat, kperm, kbias)

    out_pad = _scatter_add(
        src_pad,
        transformed,
        jnp.zeros((ZCHUNK, ATOM_DIM), jnp.float32),
    )
    return out_pad[:N_NODES]


# speedup vs baseline: 4.4947x; 1.0110x over previous
import functools

import jax
import jax.numpy as jnp
from jax import lax
from jax.experimental import pallas as pl
from jax.experimental.pallas import tpu as pltpu
from jax.experimental.pallas import tpu_sc as plsc

N_NODES = 50000
ATOM_DIM = 32
BOND_DIM = 16
M_DIM = BOND_DIM * ATOM_DIM
PACK = 128 // ATOM_DIM

NC = 2
NT = 16
NW = NC * NT
CHUNK = 128

CPW = 25
EDGES_PW = CPW * CHUNK
E_PAD = NW * EDGES_PW
EP4 = E_PAD // PACK

BLK = 4096

HALF = 25088
OUT_PAD = 2 * HALF
DRAIN_PT = HALF // NT
ACC_ROWS = 25600
ZCHUNK = ACC_ROWS // NT
DUMMY = HALF
EPT = E_PAD // NT
SCHUNKS = 5
SSTEP = SCHUNKS * CHUNK
OUTER = EPT // SSTEP
SENTINEL = 1 << 30

_mesh = plsc.VectorSubcoreMesh(core_axis_name="c", subcore_axis_name="s")


@functools.partial(
    pl.kernel,
    out_type=jax.ShapeDtypeStruct((E_PAD, 128), jnp.float32),
    mesh=_mesh,
    scratch_types=[
        pltpu.VMEM((EDGES_PW,), jnp.int32),
        pltpu.VMEM((CPW, CHUNK, ATOM_DIM), jnp.float32),
        pltpu.SemaphoreType.DMA,
        pltpu.SemaphoreType.DMA,
    ],
    compiler_params=pltpu.CompilerParams(use_tc_tiling_on_sc=False),
)
def _gather_rows(dst_hbm, atom_hbm, out_hbm, idx_v, rows_v, gsem, osem):
    wid = lax.axis_index("s") * NC + lax.axis_index("c")
    base = wid * EDGES_PW
    pltpu.sync_copy(dst_hbm.at[pl.ds(base, EDGES_PW)], idx_v)
    gathers = [
        pltpu.async_copy(
            atom_hbm.at[idx_v.at[pl.ds(j * CHUNK, CHUNK)]], rows_v.at[j], gsem
        )
        for j in range(CPW)
    ]
    stores = []
    for j in range(CPW):
        gathers[j].wait()
        stores.append(
            pltpu.async_copy(
                rows_v.at[j],
                out_hbm.at[pl.ds(base + j * CHUNK, CHUNK), pl.ds(0, ATOM_DIM)],
                osem,
            )
        )
    for d in stores:
        d.wait()


def _dense_body(nbp_ref, bond_ref, rmat_ref, tmat_ref, kperm_ref, kbias_ref, out_ref):
    nb = nbp_ref[...][:, :ATOM_DIM].astype(jnp.bfloat16)
    a = jnp.dot(
        bond_ref[...].astype(jnp.bfloat16),
        rmat_ref[...],
        preferred_element_type=jnp.float32,
    ).astype(jnp.bfloat16)
    b = jnp.dot(
        nb, tmat_ref[...], preferred_element_type=jnp.float32
    ).astype(jnp.bfloat16)
    out_ref[:, :ATOM_DIM] = jnp.dot(
        a * b, kperm_ref[...], preferred_element_type=jnp.float32
    ) + jnp.dot(nb, kbias_ref[...], preferred_element_type=jnp.float32)


@functools.partial(
    pl.kernel,
    out_type=jax.ShapeDtypeStruct((OUT_PAD, ATOM_DIM), jnp.float32),
    mesh=_mesh,
    scratch_types=[
        pltpu.VMEM((2, SSTEP), jnp.int32),
        pltpu.VMEM((SCHUNKS, CHUNK), jnp.int32),
        pltpu.VMEM((2, SSTEP, ATOM_DIM), jnp.float32),
        pltpu.VMEM_SHARED((ACC_ROWS, ATOM_DIM), jnp.float32),
        pltpu.SemaphoreType.DMA,
        pltpu.SemaphoreType.DMA,
    ],
    compiler_params=pltpu.CompilerParams(use_tc_tiling_on_sc=False),
)
def _scatter_add(src_hbm, t_hbm, zero_hbm, out_hbm, sidx_v, lidx_v, tv, acc, lsem, ssem):
    c = lax.axis_index("c")
    s = lax.axis_index("s")

    def load(o):
        ebase = s * EPT + o * SSTEP
        return (
            pltpu.async_copy(src_hbm.at[pl.ds(ebase, SSTEP)], sidx_v.at[o % 2], lsem),
            pltpu.async_copy(
                t_hbm.at[pl.ds(ebase, SSTEP), pl.ds(0, ATOM_DIM)], tv.at[o % 2], lsem
            ),
        )

    pending = load(0)
    pltpu.sync_copy(zero_hbm, acc.at[pl.ds(s * ZCHUNK, ZCHUNK)])
    plsc.subcore_barrier()
    base = c * HALF

    for o in range(OUTER):
        b = o % 2
        for d in pending:
            d.wait()
        if o + 1 < OUTER:
            pending = load(o + 1)
        for r in range(SCHUNKS):
            for q in range(CHUNK // 16):
                v = sidx_v[b, pl.ds(r * CHUNK + q * 16, 16)]
                il = v - base
                ok = (il >= 0) & (il < HALF)
                lidx_v[r, pl.ds(q * 16, 16)] = jnp.where(ok, il, DUMMY)
        scatters = [
            pltpu.async_copy(
                tv.at[b, pl.ds(r * CHUNK, CHUNK)], acc.at[lidx_v.at[r]], ssem, add=True
            )
            for r in range(SCHUNKS)
        ]
        for d in scatters:
            d.wait()
    plsc.subcore_barrier()
    pltpu.sync_copy(
        acc.at[pl.ds(s * DRAIN_PT, DRAIN_PT)],
        out_hbm.at[pl.ds(c * HALF + s * DRAIN_PT, DRAIN_PT)],
    )


def kernel(atom_features, bond_features, pair_indices, kernel, bias):
    pi = pair_indices.astype(jnp.int32)
    src, dst = pi[:, 0], pi[:, 1]
    n_edges = src.shape[0]
    pad = E_PAD - n_edges

    dst_pad = jnp.concatenate([dst, jnp.zeros((pad,), jnp.int32)])
    src_pad = jnp.concatenate([src, jnp.full((pad,), SENTINEL, jnp.int32)])

    kperm = kernel.reshape(BOND_DIM, ATOM_DIM, ATOM_DIM).transpose(0, 2, 1).reshape(
        M_DIM, ATOM_DIM
    )
    kbias = bias.reshape(ATOM_DIM, ATOM_DIM).T
    kperm = kperm.astype(jnp.bfloat16)
    rmat = jnp.repeat(jnp.eye(BOND_DIM, dtype=jnp.bfloat16), ATOM_DIM, axis=1)
    tmat = jnp.tile(jnp.eye(ATOM_DIM, dtype=jnp.bfloat16), (1, BOND_DIM))

    nb_packed = _gather_rows(dst_pad, atom_features)

    n_blocks = (n_edges + BLK - 1) // BLK
    transformed = pl.pallas_call(
        _dense_body,
        grid=(n_blocks,),
        in_specs=[
            pl.BlockSpec((BLK, 128), lambda i: (i, 0)),
            pl.BlockSpec((BLK, BOND_DIM), lambda i: (i, 0)),
            pl.BlockSpec((BOND_DIM, M_DIM), lambda i: (0, 0)),
            pl.BlockSpec((ATOM_DIM, M_DIM), lambda i: (0, 0)),
            pl.BlockSpec((M_DIM, ATOM_DIM), lambda i: (0, 0)),
            pl.BlockSpec((ATOM_DIM, ATOM_DIM), lambda i: (0, 0)),
        ],
        out_specs=pl.BlockSpec((BLK, 128), lambda i: (i, 0)),
        out_shape=jax.ShapeDtypeStruct((E_PAD, 128), jnp.float32),
        compiler_params=pltpu.CompilerParams(
            dimension_semantics=("arbitrary",)
        ),
    )(nb_packed, bond_features, rmat, tmat, kperm, kbias)

    out_pad = _scatter_add(
        src_pad,
        transformed,
        jnp.zeros((ZCHUNK, ATOM_DIM), jnp.float32),
    )
    return out_pad[:N_NODES]


# speedup vs baseline: 5.2399x; 1.1658x over previous
import functools

import jax
import jax.numpy as jnp
from jax import lax
from jax.experimental import pallas as pl
from jax.experimental.pallas import tpu as pltpu
from jax.experimental.pallas import tpu_sc as plsc

N_NODES = 50000
ATOM_DIM = 32
BOND_DIM = 16
M_DIM = BOND_DIM * ATOM_DIM
PACK = 128 // ATOM_DIM

NC = 2
NT = 16
NW = NC * NT
CHUNK = 128

CPW = 25
EDGES_PW = CPW * CHUNK
E_PAD = NW * EDGES_PW
EP4 = E_PAD // PACK

BLK = 4096

HALF = 25088
OUT_PAD = 2 * HALF
DRAIN_PT = HALF // NT
ACC_ROWS = 25600
ZCHUNK = ACC_ROWS // NT
DUMMY = HALF
EPT = E_PAD // NT
SCHUNKS = 5
SSTEP = SCHUNKS * CHUNK
OUTER = EPT // SSTEP
SENTINEL = 1 << 30

_mesh = plsc.VectorSubcoreMesh(core_axis_name="c", subcore_axis_name="s")


@functools.partial(
    pl.kernel,
    out_type=jax.ShapeDtypeStruct((E_PAD, 128), jnp.float32),
    mesh=_mesh,
    scratch_types=[
        pltpu.VMEM((EDGES_PW,), jnp.int32),
        pltpu.VMEM((CPW, CHUNK, ATOM_DIM), jnp.float32),
        pltpu.SemaphoreType.DMA,
        pltpu.SemaphoreType.DMA,
    ],
    compiler_params=pltpu.CompilerParams(use_tc_tiling_on_sc=False),
)
def _gather_rows(dst_hbm, atom_hbm, out_hbm, idx_v, rows_v, gsem, osem):
    wid = lax.axis_index("s") * NC + lax.axis_index("c")
    base = wid * EDGES_PW
    pltpu.sync_copy(dst_hbm.at[pl.ds(base, EDGES_PW)], idx_v)
    gathers = [
        pltpu.async_copy(
            atom_hbm.at[idx_v.at[pl.ds(j * CHUNK, CHUNK)]], rows_v.at[j], gsem
        )
        for j in range(CPW)
    ]
    stores = []
    for j in range(CPW):
        gathers[j].wait()
        stores.append(
            pltpu.async_copy(
                rows_v.at[j],
                out_hbm.at[pl.ds(base + j * CHUNK, CHUNK), pl.ds(0, ATOM_DIM)],
                osem,
            )
        )
    for d in stores:
        d.wait()


def _dense_body(nbp_ref, bond_ref, rmat_ref, tmat_ref, kperm_ref, kbias_ref, out_ref):
    nb = nbp_ref[...][:, :ATOM_DIM].astype(jnp.bfloat16)
    a = jnp.dot(
        bond_ref[...].astype(jnp.bfloat16),
        rmat_ref[...],
        preferred_element_type=jnp.float32,
    ).astype(jnp.bfloat16)
    b = jnp.dot(
        nb, tmat_ref[...], preferred_element_type=jnp.float32
    ).astype(jnp.bfloat16)
    out_ref[:, :ATOM_DIM] = jnp.dot(
        a * b, kperm_ref[...], preferred_element_type=jnp.float32
    ) + jnp.dot(nb, kbias_ref[...], preferred_element_type=jnp.float32)


@functools.partial(
    pl.kernel,
    out_type=jax.ShapeDtypeStruct((OUT_PAD, ATOM_DIM), jnp.float32),
    mesh=_mesh,
    scratch_types=[
        pltpu.VMEM((2, SSTEP), jnp.int32),
        pltpu.VMEM((SCHUNKS, CHUNK), jnp.int32),
        pltpu.VMEM((2, SSTEP, ATOM_DIM), jnp.float32),
        pltpu.VMEM_SHARED((ACC_ROWS, ATOM_DIM), jnp.float32),
        pltpu.SemaphoreType.DMA,
        pltpu.SemaphoreType.DMA,
    ],
    compiler_params=pltpu.CompilerParams(use_tc_tiling_on_sc=False),
)
def _scatter_add(src_hbm, t_hbm, zero_hbm, out_hbm, sidx_v, lidx_v, tv, acc, lsem, ssem):
    c = lax.axis_index("c")
    s = lax.axis_index("s")

    def load(o):
        ebase = s * EPT + o * SSTEP
        return (
            pltpu.async_copy(src_hbm.at[pl.ds(ebase, SSTEP)], sidx_v.at[o % 2], lsem),
            pltpu.async_copy(
                t_hbm.at[pl.ds(ebase, SSTEP), pl.ds(0, ATOM_DIM)], tv.at[o % 2], lsem
            ),
        )

    pending = load(0)
    pltpu.sync_copy(zero_hbm, acc.at[pl.ds(s * ZCHUNK, ZCHUNK)])
    plsc.subcore_barrier()
    base = c * HALF

    for o in range(OUTER):
        b = o % 2
        for d in pending:
            d.wait()
        if o + 1 < OUTER:
            pending = load(o + 1)
        for r in range(SCHUNKS):
            for q in range(CHUNK // 16):
                v = sidx_v[b, pl.ds(r * CHUNK + q * 16, 16)]
                il = v - base
                ok = (il >= 0) & (il < HALF)
                dummy = DUMMY + ((r * CHUNK + q * 16) & 511) + lax.iota(jnp.int32, 16)
                lidx_v[r, pl.ds(q * 16, 16)] = jnp.where(ok, il, dummy)
        scatters = [
            pltpu.async_copy(
                tv.at[b, pl.ds(r * CHUNK, CHUNK)], acc.at[lidx_v.at[r]], ssem, add=True
            )
            for r in range(SCHUNKS)
        ]
        for d in scatters:
            d.wait()
    plsc.subcore_barrier()
    pltpu.sync_copy(
        acc.at[pl.ds(s * DRAIN_PT, DRAIN_PT)],
        out_hbm.at[pl.ds(c * HALF + s * DRAIN_PT, DRAIN_PT)],
    )


def kernel(atom_features, bond_features, pair_indices, kernel, bias):
    pi = pair_indices.astype(jnp.int32)
    src, dst = pi[:, 0], pi[:, 1]
    n_edges = src.shape[0]
    pad = E_PAD - n_edges

    dst_pad = jnp.concatenate([dst, jnp.zeros((pad,), jnp.int32)])
    src_pad = jnp.concatenate([src, jnp.full((pad,), SENTINEL, jnp.int32)])

    kperm = kernel.reshape(BOND_DIM, ATOM_DIM, ATOM_DIM).transpose(0, 2, 1).reshape(
        M_DIM, ATOM_DIM
    )
    kbias = bias.reshape(ATOM_DIM, ATOM_DIM).T
    kperm = kperm.astype(jnp.bfloat16)
    rmat = jnp.repeat(jnp.eye(BOND_DIM, dtype=jnp.bfloat16), ATOM_DIM, axis=1)
    tmat = jnp.tile(jnp.eye(ATOM_DIM, dtype=jnp.bfloat16), (1, BOND_DIM))

    nb_packed = _gather_rows(dst_pad, atom_features)

    n_blocks = (n_edges + BLK - 1) // BLK
    transformed = pl.pallas_call(
        _dense_body,
        grid=(n_blocks,),
        in_specs=[
            pl.BlockSpec((BLK, 128), lambda i: (i, 0)),
            pl.BlockSpec((BLK, BOND_DIM), lambda i: (i, 0)),
            pl.BlockSpec((BOND_DIM, M_DIM), lambda i: (0, 0)),
            pl.BlockSpec((ATOM_DIM, M_DIM), lambda i: (0, 0)),
            pl.BlockSpec((M_DIM, ATOM_DIM), lambda i: (0, 0)),
            pl.BlockSpec((ATOM_DIM, ATOM_DIM), lambda i: (0, 0)),
        ],
        out_specs=pl.BlockSpec((BLK, 128), lambda i: (i, 0)),
        out_shape=jax.ShapeDtypeStruct((E_PAD, 128), jnp.float32),
        compiler_params=pltpu.CompilerParams(
            dimension_semantics=("arbitrary",)
        ),
    )(nb_packed, bond_features, rmat, tmat, kperm, kbias)

    out_pad = _scatter_add(
        src_pad,
        transformed,
        jnp.zeros((ZCHUNK, ATOM_DIM), jnp.float32),
    )
    return out_pad[:N_NODES]


# speedup vs baseline: 5.8634x; 1.1190x over previous
import functools

import jax
import jax.numpy as jnp
from jax import lax
from jax.experimental import pallas as pl
from jax.experimental.pallas import tpu as pltpu
from jax.experimental.pallas import tpu_sc as plsc

N_NODES = 50000
ATOM_DIM = 32
BOND_DIM = 16
M_DIM = BOND_DIM * ATOM_DIM
PACK = 128 // ATOM_DIM

NC = 2
NT = 16
NW = NC * NT
CHUNK = 128

CPW = 25
EDGES_PW = CPW * CHUNK
E_PAD = NW * EDGES_PW
EP4 = E_PAD // PACK

BLK = 4096

HALF = 25088
OUT_PAD = 2 * HALF
DRAIN_PT = HALF // NT
ACC_ROWS = 25600
ZCHUNK = ACC_ROWS // NT
DUMMY = HALF
EPT = E_PAD // NT
SCHUNKS = 5
SSTEP = SCHUNKS * CHUNK
OUTER = EPT // SSTEP
SENTINEL = 1 << 30

_mesh = plsc.VectorSubcoreMesh(core_axis_name="c", subcore_axis_name="s")


@functools.partial(
    pl.kernel,
    out_type=jax.ShapeDtypeStruct((E_PAD, 128), jnp.float32),
    mesh=_mesh,
    scratch_types=[
        pltpu.VMEM((EDGES_PW,), jnp.int32),
        pltpu.VMEM((CPW, CHUNK, ATOM_DIM), jnp.float32),
        pltpu.SemaphoreType.DMA,
        pltpu.SemaphoreType.DMA,
    ],
    compiler_params=pltpu.CompilerParams(use_tc_tiling_on_sc=False),
)
def _gather_rows(dst_hbm, atom_hbm, out_hbm, idx_v, rows_v, gsem, osem):
    wid = lax.axis_index("s") * NC + lax.axis_index("c")
    base = wid * EDGES_PW
    pltpu.sync_copy(dst_hbm.at[pl.ds(base, EDGES_PW)], idx_v)
    gathers = [
        pltpu.async_copy(
            atom_hbm.at[idx_v.at[pl.ds(j * CHUNK, CHUNK)]], rows_v.at[j], gsem
        )
        for j in range(CPW)
    ]
    stores = []
    for j in range(CPW):
        gathers[j].wait()
        stores.append(
            pltpu.async_copy(
                rows_v.at[j],
                out_hbm.at[pl.ds(base + j * CHUNK, CHUNK), pl.ds(0, ATOM_DIM)],
                osem,
            )
        )
    for d in stores:
        d.wait()


def _dense_body(nbp_ref, bondT_ref, rmat_ref, tmat_ref, kperm_ref, kbias_ref, out_ref):
    nb = nbp_ref[...][:, :ATOM_DIM].astype(jnp.bfloat16)
    a = lax.dot_general(
        bondT_ref[...].astype(jnp.bfloat16),
        rmat_ref[...],
        (((0,), (0,)), ((), ())),
        preferred_element_type=jnp.float32,
    ).astype(jnp.bfloat16)
    b = jnp.dot(
        nb, tmat_ref[...], preferred_element_type=jnp.float32
    ).astype(jnp.bfloat16)
    out_ref[:, :ATOM_DIM] = jnp.dot(
        a * b, kperm_ref[...], preferred_element_type=jnp.float32
    ) + jnp.dot(nb, kbias_ref[...], preferred_element_type=jnp.float32)


@functools.partial(
    pl.kernel,
    out_type=jax.ShapeDtypeStruct((N_NODES, ATOM_DIM), jnp.float32),
    mesh=_mesh,
    scratch_types=[
        pltpu.VMEM((2, SSTEP), jnp.int32),
        pltpu.VMEM((SCHUNKS, CHUNK), jnp.int32),
        pltpu.VMEM((2, SSTEP, ATOM_DIM), jnp.float32),
        pltpu.VMEM_SHARED((ACC_ROWS, ATOM_DIM), jnp.float32),
        pltpu.SemaphoreType.DMA,
        pltpu.SemaphoreType.DMA,
    ],
    compiler_params=pltpu.CompilerParams(use_tc_tiling_on_sc=False),
)
def _scatter_add(src_hbm, t_hbm, zero_hbm, out_hbm, sidx_v, lidx_v, tv, acc, lsem, ssem):
    c = lax.axis_index("c")
    s = lax.axis_index("s")

    def load(o):
        ebase = s * EPT + o * SSTEP
        return (
            pltpu.async_copy(src_hbm.at[pl.ds(ebase, SSTEP)], sidx_v.at[o % 2], lsem),
            pltpu.async_copy(
                t_hbm.at[pl.ds(ebase, SSTEP), pl.ds(0, ATOM_DIM)], tv.at[o % 2], lsem
            ),
        )

    pending = load(0)
    pltpu.sync_copy(zero_hbm, acc.at[pl.ds(s * ZCHUNK, ZCHUNK)])
    plsc.subcore_barrier()
    base = c * HALF

    for o in range(OUTER):
        b = o % 2
        for d in pending:
            d.wait()
        if o + 1 < OUTER:
            pending = load(o + 1)
        for r in range(SCHUNKS):
            for q in range(CHUNK // 16):
                v = sidx_v[b, pl.ds(r * CHUNK + q * 16, 16)]
                il = v - base
                ok = (il >= 0) & (il < HALF)
                dummy = DUMMY + ((r * CHUNK + q * 16) & 511) + lax.iota(jnp.int32, 16)
                lidx_v[r, pl.ds(q * 16, 16)] = jnp.where(ok, il, dummy)
        scatters = [
            pltpu.async_copy(
                tv.at[b, pl.ds(r * CHUNK, CHUNK)], acc.at[lidx_v.at[r]], ssem, add=True
            )
            for r in range(SCHUNKS)
        ]
        for d in scatters:
            d.wait()
    plsc.subcore_barrier()
    out_start = jnp.minimum(c * HALF + s * DRAIN_PT, N_NODES - DRAIN_PT)
    pltpu.sync_copy(
        acc.at[pl.ds(out_start - c * HALF, DRAIN_PT)],
        out_hbm.at[pl.ds(out_start, DRAIN_PT)],
    )


def kernel(atom_features, bond_features, pair_indices, kernel, bias):
    pi = pair_indices.astype(jnp.int32)
    src, dst = pi[:, 0], pi[:, 1]
    n_edges = src.shape[0]
    pad = E_PAD - n_edges

    dst_pad = jnp.concatenate([dst, jnp.zeros((pad,), jnp.int32)])
    src_pad = jnp.concatenate([src, jnp.full((pad,), SENTINEL, jnp.int32)])

    kperm = kernel.reshape(BOND_DIM, ATOM_DIM, ATOM_DIM).transpose(0, 2, 1).reshape(
        M_DIM, ATOM_DIM
    )
    kbias = bias.reshape(ATOM_DIM, ATOM_DIM).T
    kperm = kperm.astype(jnp.bfloat16)
    rmat = jnp.repeat(jnp.eye(BOND_DIM, dtype=jnp.bfloat16), ATOM_DIM, axis=1)
    tmat = jnp.tile(jnp.eye(ATOM_DIM, dtype=jnp.bfloat16), (1, BOND_DIM))

    nb_packed = _gather_rows(dst_pad, atom_features)

    n_blocks = (n_edges + BLK - 1) // BLK
    transformed = pl.pallas_call(
        _dense_body,
        grid=(n_blocks,),
        in_specs=[
            pl.BlockSpec((BLK, 128), lambda i: (i, 0)),
            pl.BlockSpec((BOND_DIM, BLK), lambda i: (0, i)),
            pl.BlockSpec((BOND_DIM, M_DIM), lambda i: (0, 0)),
            pl.BlockSpec((ATOM_DIM, M_DIM), lambda i: (0, 0)),
            pl.BlockSpec((M_DIM, ATOM_DIM), lambda i: (0, 0)),
            pl.BlockSpec((ATOM_DIM, ATOM_DIM), lambda i: (0, 0)),
        ],
        out_specs=pl.BlockSpec((BLK, 128), lambda i: (i, 0)),
        out_shape=jax.ShapeDtypeStruct((E_PAD, 128), jnp.float32),
        compiler_params=pltpu.CompilerParams(
            dimension_semantics=("arbitrary",)
        ),
    )(nb_packed, bond_features.T, rmat, tmat, kperm, kbias)

    return _scatter_add(
        src_pad,
        transformed,
        jnp.zeros((ZCHUNK, ATOM_DIM), jnp.float32),
    )
